# baseline (speedup 1.0000x reference)
import functools

import jax
import jax.numpy as jnp
from jax import lax
from jax.experimental import pallas as pl
from jax.experimental.pallas import tpu as pltpu
from jax.experimental.pallas import tpu_sc as plsc

_N_TYPES = 2
_N_PAIR_TYPES = 3
_BATCH, _N_ATOMS, _N_NEIGH = 16, 512, 64

_K_SC = 2
_B_TC = _BATCH - _K_SC

_L = 16
_NC, _NS = 2, 16
_NW = _NC * _NS
_W_PER_B = _NW // _K_SC
_ATOMS_PER_W = _N_ATOMS // _W_PER_B
_GROUPS = _ATOMS_PER_W // _L
_CHUNKS = _N_NEIGH // _L


def _vsqrt(x):
    xi = lax.bitcast_convert_type(x, jnp.int32)
    seed = jnp.full((_L,), 0x5F3759DF, jnp.int32) - lax.shift_right_arithmetic(
        xi, jnp.full((_L,), 1, jnp.int32))
    y = lax.bitcast_convert_type(seed, jnp.float32)
    half, three_half = jnp.float32(0.5), jnp.float32(1.5)
    for _ in range(3):
        y = y * (three_half - half * x * y * y)
    return x * y


def _sc_body(d_hbm, pt_hbm, ty_hbm, par_hbm, out_hbm, d_v, pt_v, ty_v, par_v, res_v):
    wid = lax.axis_index("s") * _NC + lax.axis_index("c")
    b = wid // _W_PER_B
    slot = wid % _W_PER_B
    a0 = slot * _ATOMS_PER_W
    pltpu.sync_copy(d_hbm.at[b, pl.ds(a0, _ATOMS_PER_W), :], d_v)
    pltpu.sync_copy(pt_hbm.at[b, pl.ds(a0, _ATOMS_PER_W), :], pt_v)
    pltpu.sync_copy(ty_hbm.at[b, pl.ds(a0, _ATOMS_PER_W)], ty_v)
    pltpu.sync_copy(par_hbm, par_v)

    pv = par_v[...]

    def bcast(i):
        return jnp.full((_L,), pv[i], jnp.float32)

    lnA = [bcast(0), bcast(1), bcast(2)]
    pp = [bcast(3), bcast(4), bcast(5)]
    lnX = [bcast(6), bcast(7), bcast(8)]
    qq = [bcast(9), bcast(10), bcast(11)]
    d0v, d1v = bcast(12), bcast(13)

    iota = lax.iota(jnp.int32, _L)
    zero = jnp.zeros((_L,), jnp.float32)
    one = jnp.full((_L,), 1, jnp.int32)
    two = jnp.full((_L,), 2, jnp.int32)

    def chunk(a, k, pacc, racc):
        d = d_v[a, pl.ds(k * _L, _L)]
        ptv = pt_v[a, pl.ds(k * _L, _L)]
        m1 = ptv == one
        m2 = ptv == two
        la = jnp.where(m1, lnA[1], jnp.where(m2, lnA[2], lnA[0]))
        p = jnp.where(m1, pp[1], jnp.where(m2, pp[2], pp[0]))
        lx = jnp.where(m1, lnX[1], jnp.where(m2, lnX[2], lnX[0]))
        q = jnp.where(m1, qq[1], jnp.where(m2, qq[2], qq[0]))
        pacc = pacc + jnp.exp(la - p * d)
        racc = racc + jnp.exp(lx - q * d)
        return pacc, racc

    def group_body(g, carry):
        acc_phi, acc_emb = carry

        def atom_body(a, carry2):
            acc2, m = carry2
            pacc, racc = zero, zero
            for k in range(_CHUNKS):
                pacc, racc = chunk(g * _L + a, k, pacc, racc)
            tot = jnp.sum(racc)
            m = jnp.where(iota == a, tot, m)
            return acc2 + pacc, m

        acc_phi, m = lax.fori_loop(0, _L, atom_body, (acc_phi, zero))
        sq = _vsqrt(m)
        tyv = ty_v[pl.ds(pl.multiple_of(g * _L, _L), _L)]
        dsel = jnp.where(tyv == one, d1v, d0v)
        acc_emb = acc_emb - dsel * sq
        return acc_phi, acc_emb

    acc_phi, acc_emb = lax.fori_loop(0, _GROUPS, group_body, (zero, zero))
    res_v[...] = (acc_phi + acc_emb) * jnp.float32(1.0 / _N_ATOMS)
    pltpu.sync_copy(res_v, out_hbm.at[b, pl.ds(slot * _L, _L)])


def _sc_call(d3, pt3, ty2, par):
    mesh = plsc.VectorSubcoreMesh(core_axis_name="c", subcore_axis_name="s")
    run = functools.partial(
        pl.kernel,
        mesh=mesh,
        compiler_params=pltpu.CompilerParams(needs_layout_passes=False),
        out_type=jax.ShapeDtypeStruct((_K_SC, _W_PER_B * _L), jnp.float32),
        scratch_types=[
            pltpu.VMEM((_ATOMS_PER_W, _N_NEIGH), jnp.float32),
            pltpu.VMEM((_ATOMS_PER_W, _N_NEIGH), jnp.int32),
            pltpu.VMEM((_ATOMS_PER_W,), jnp.int32),
            pltpu.VMEM((_L,), jnp.float32),
            pltpu.VMEM((_L,), jnp.float32),
        ],
    )(_sc_body)
    return run(d3, pt3, ty2, par)


def _tc_body(par_ref, d_ref, pt_ref, ty_ref, out_ref):
    d = d_ref[0]
    pt = pt_ref[0]
    m1 = pt == 1
    m2 = pt == 2
    la = jnp.where(m1, par_ref[1], jnp.where(m2, par_ref[2], par_ref[0]))
    p = jnp.where(m1, par_ref[4], jnp.where(m2, par_ref[5], par_ref[3]))
    lx = jnp.where(m1, par_ref[7], jnp.where(m2, par_ref[8], par_ref[6]))
    q = jnp.where(m1, par_ref[10], jnp.where(m2, par_ref[11], par_ref[9]))
    phi = jnp.exp(la - p * d)
    rho = jnp.exp(lx - q * d)
    srho = jnp.sum(rho, axis=-1, keepdims=True)
    ty = ty_ref[0]
    dsel = jnp.where(ty == 1, par_ref[13], par_ref[12])
    emb = -dsel * jnp.sqrt(srho)
    tot = (jnp.sum(phi) + jnp.sum(emb)) * (1.0 / _N_ATOMS)
    out_ref[...] = jnp.full((1, 1, 1), tot, jnp.float32)


def _tc_call(par, d3, pt3, ty3):
    return pl.pallas_call(
        _tc_body,
        grid=(_B_TC,),
        in_specs=[
            pl.BlockSpec(memory_space=pltpu.SMEM),
            pl.BlockSpec((1, _N_ATOMS, _N_NEIGH), lambda b: (b, 0, 0)),
            pl.BlockSpec((1, _N_ATOMS, _N_NEIGH), lambda b: (b, 0, 0)),
            pl.BlockSpec((1, _N_ATOMS, 1), lambda b: (b, 0, 0)),
        ],
        out_specs=pl.BlockSpec((1, 1, 1), lambda b: (b, 0, 0)),
        out_shape=jax.ShapeDtypeStruct((_B_TC, 1, 1), jnp.float32),
    )(par, d3, pt3, ty3)


@jax.jit
def _eam_hybrid(distances, pair_types, types, par):
    tc_e = _tc_call(par, distances[:_B_TC], pair_types[:_B_TC],
                    types[:_B_TC].reshape(_B_TC, _N_ATOMS, 1))
    sc_partials = _sc_call(distances[_B_TC:], pair_types[_B_TC:],
                           types[_B_TC:], par)
    sc_e = sc_partials.sum(axis=1, keepdims=True)
    return jnp.concatenate([tc_e.reshape(_B_TC, 1), sc_e], axis=0)


def kernel(types, distances, pair_types, phi_params, rho_params, emb_params):
    par = jnp.concatenate([
        jnp.log(phi_params[:, 0]), phi_params[:, 1],
        jnp.log(rho_params[:, 0]), rho_params[:, 1],
        emb_params.astype(jnp.float32), jnp.zeros((2,), jnp.float32),
    ]).astype(jnp.float32)
    return _eam_hybrid(distances, pair_types.astype(jnp.int32),
                       types.astype(jnp.int32), par)


# speedup vs baseline: 1.0136x; 1.0136x over previous
import functools

import jax
import jax.numpy as jnp
from jax import lax
from jax.experimental import pallas as pl
from jax.experimental.pallas import tpu as pltpu
from jax.experimental.pallas import tpu_sc as plsc

_N_TYPES = 2
_N_PAIR_TYPES = 3
_BATCH, _N_ATOMS, _N_NEIGH = 16, 512, 64

_K_SC = 4
_B_TC = _BATCH - _K_SC

_L = 16
_NC, _NS = 2, 16
_NW = _NC * _NS
_W_PER_B = _NW // _K_SC
_ATOMS_PER_W = _N_ATOMS // _W_PER_B
_GROUPS = _ATOMS_PER_W // _L
_CHUNKS = _N_NEIGH // _L


def _vsqrt(x):
    xi = lax.bitcast_convert_type(x, jnp.int32)
    seed = jnp.full((_L,), 0x5F3759DF, jnp.int32) - lax.shift_right_arithmetic(
        xi, jnp.full((_L,), 1, jnp.int32))
    y = lax.bitcast_convert_type(seed, jnp.float32)
    half, three_half = jnp.float32(0.5), jnp.float32(1.5)
    for _ in range(3):
        y = y * (three_half - half * x * y * y)
    return x * y


def _sc_body(d_hbm, pt_hbm, ty_hbm, par_hbm, out_hbm, d_v, pt_v, ty_v, par_v, res_v):
    wid = lax.axis_index("s") * _NC + lax.axis_index("c")
    b = wid // _W_PER_B
    slot = wid % _W_PER_B
    a0 = slot * _ATOMS_PER_W
    pltpu.sync_copy(d_hbm.at[b, pl.ds(a0, _ATOMS_PER_W), :], d_v)
    pltpu.sync_copy(pt_hbm.at[b, pl.ds(a0, _ATOMS_PER_W), :], pt_v)
    pltpu.sync_copy(ty_hbm.at[b, pl.ds(a0, _ATOMS_PER_W)], ty_v)
    pltpu.sync_copy(par_hbm, par_v)

    pv = par_v[...]

    def bcast(i):
        return jnp.full((_L,), pv[i], jnp.float32)

    lnA = [bcast(0), bcast(1), bcast(2)]
    pp = [bcast(3), bcast(4), bcast(5)]
    lnX = [bcast(6), bcast(7), bcast(8)]
    qq = [bcast(9), bcast(10), bcast(11)]
    d0v, d1v = bcast(12), bcast(13)

    iota = lax.iota(jnp.int32, _L)
    zero = jnp.zeros((_L,), jnp.float32)
    one = jnp.full((_L,), 1, jnp.int32)
    two = jnp.full((_L,), 2, jnp.int32)

    def chunk(a, k, pacc, racc):
        d = d_v[a, pl.ds(k * _L, _L)]
        ptv = pt_v[a, pl.ds(k * _L, _L)]
        m1 = ptv == one
        m2 = ptv == two
        la = jnp.where(m1, lnA[1], jnp.where(m2, lnA[2], lnA[0]))
        p = jnp.where(m1, pp[1], jnp.where(m2, pp[2], pp[0]))
        lx = jnp.where(m1, lnX[1], jnp.where(m2, lnX[2], lnX[0]))
        q = jnp.where(m1, qq[1], jnp.where(m2, qq[2], qq[0]))
        pacc = pacc + jnp.exp(la - p * d)
        racc = racc + jnp.exp(lx - q * d)
        return pacc, racc

    def group_body(g, carry):
        acc_phi, acc_emb = carry

        def atom_body(a, carry2):
            acc2, m = carry2
            pacc, racc = zero, zero
            for k in range(_CHUNKS):
                pacc, racc = chunk(g * _L + a, k, pacc, racc)
            tot = jnp.sum(racc)
            m = jnp.where(iota == a, tot, m)
            return acc2 + pacc, m

        acc_phi, m = lax.fori_loop(0, _L, atom_body, (acc_phi, zero))
        sq = _vsqrt(m)
        tyv = ty_v[pl.ds(pl.multiple_of(g * _L, _L), _L)]
        dsel = jnp.where(tyv == one, d1v, d0v)
        acc_emb = acc_emb - dsel * sq
        return acc_phi, acc_emb

    acc_phi, acc_emb = lax.fori_loop(0, _GROUPS, group_body, (zero, zero))
    res_v[...] = (acc_phi + acc_emb) * jnp.float32(1.0 / _N_ATOMS)
    pltpu.sync_copy(res_v, out_hbm.at[b, pl.ds(slot * _L, _L)])


def _sc_call(d3, pt3, ty2, par):
    mesh = plsc.VectorSubcoreMesh(core_axis_name="c", subcore_axis_name="s")
    run = functools.partial(
        pl.kernel,
        mesh=mesh,
        compiler_params=pltpu.CompilerParams(needs_layout_passes=False),
        out_type=jax.ShapeDtypeStruct((_K_SC, _W_PER_B * _L), jnp.float32),
        scratch_types=[
            pltpu.VMEM((_ATOMS_PER_W, _N_NEIGH), jnp.float32),
            pltpu.VMEM((_ATOMS_PER_W, _N_NEIGH), jnp.int32),
            pltpu.VMEM((_ATOMS_PER_W,), jnp.int32),
            pltpu.VMEM((_L,), jnp.float32),
            pltpu.VMEM((_L,), jnp.float32),
        ],
    )(_sc_body)
    return run(d3, pt3, ty2, par)


def _tc_body(par_ref, d_ref, pt_ref, ty_ref, out_ref):
    d = d_ref[0]
    pt = pt_ref[0]
    m1 = pt == 1
    m2 = pt == 2
    la = jnp.where(m1, par_ref[1], jnp.where(m2, par_ref[2], par_ref[0]))
    p = jnp.where(m1, par_ref[4], jnp.where(m2, par_ref[5], par_ref[3]))
    lx = jnp.where(m1, par_ref[7], jnp.where(m2, par_ref[8], par_ref[6]))
    q = jnp.where(m1, par_ref[10], jnp.where(m2, par_ref[11], par_ref[9]))
    phi = jnp.exp(la - p * d)
    rho = jnp.exp(lx - q * d)
    srho = jnp.sum(rho, axis=-1, keepdims=True)
    ty = ty_ref[0]
    dsel = jnp.where(ty == 1, par_ref[13], par_ref[12])
    emb = -dsel * jnp.sqrt(srho)
    tot = (jnp.sum(phi) + jnp.sum(emb)) * (1.0 / _N_ATOMS)
    out_ref[...] = jnp.full((1, 1, 1), tot, jnp.float32)


def _tc_call(par, d3, pt3, ty3):
    return pl.pallas_call(
        _tc_body,
        grid=(_B_TC,),
        in_specs=[
            pl.BlockSpec(memory_space=pltpu.SMEM),
            pl.BlockSpec((1, _N_ATOMS, _N_NEIGH), lambda b: (b, 0, 0)),
            pl.BlockSpec((1, _N_ATOMS, _N_NEIGH), lambda b: (b, 0, 0)),
            pl.BlockSpec((1, _N_ATOMS, 1), lambda b: (b, 0, 0)),
        ],
        out_specs=pl.BlockSpec((1, 1, 1), lambda b: (b, 0, 0)),
        out_shape=jax.ShapeDtypeStruct((_B_TC, 1, 1), jnp.float32),
    )(par, d3, pt3, ty3)


@jax.jit
def _eam_hybrid(distances, pair_types, types, par):
    tc_e = _tc_call(par, distances, pair_types,
                    types.reshape(_BATCH, _N_ATOMS, 1))
    sc_partials = _sc_call(distances[_B_TC:], pair_types[_B_TC:],
                           types[_B_TC:], par)
    sc_e = sc_partials.sum(axis=1, keepdims=True)
    return jnp.concatenate([tc_e.reshape(_B_TC, 1), sc_e], axis=0)


def kernel(types, distances, pair_types, phi_params, rho_params, emb_params):
    par = jnp.concatenate([
        jnp.log(phi_params[:, 0]), phi_params[:, 1],
        jnp.log(rho_params[:, 0]), rho_params[:, 1],
        emb_params.astype(jnp.float32), jnp.zeros((2,), jnp.float32),
    ]).astype(jnp.float32)
    return _eam_hybrid(distances, pair_types.astype(jnp.int32),
                       types.astype(jnp.int32), par)


# speedup vs baseline: 1.0304x; 1.0165x over previous
import functools

import jax
import jax.numpy as jnp
from jax import lax
from jax.experimental import pallas as pl
from jax.experimental.pallas import tpu as pltpu
from jax.experimental.pallas import tpu_sc as plsc

_N_TYPES = 2
_N_PAIR_TYPES = 3
_BATCH, _N_ATOMS, _N_NEIGH = 16, 512, 64

_K_SC = 2
_B_TC = _BATCH - _K_SC

_L = 16
_NC, _NS = 2, 16
_NW = _NC * _NS
_W_PER_B = _NW // _K_SC
_ATOMS_PER_W = _N_ATOMS // _W_PER_B
_GROUPS = _ATOMS_PER_W // _L
_CHUNKS = _N_NEIGH // _L


def _vsqrt(x):
    xi = lax.bitcast_convert_type(x, jnp.int32)
    seed = jnp.full((_L,), 0x5F3759DF, jnp.int32) - lax.shift_right_arithmetic(
        xi, jnp.full((_L,), 1, jnp.int32))
    y = lax.bitcast_convert_type(seed, jnp.float32)
    half, three_half = jnp.float32(0.5), jnp.float32(1.5)
    for _ in range(3):
        y = y * (three_half - half * x * y * y)
    return x * y


def _sc_body(d_hbm, pt_hbm, ty_hbm, par_hbm, out_hbm, d_v, pt_v, ty_v, par_v, res_v):
    wid = lax.axis_index("s") * _NC + lax.axis_index("c")
    b = wid // _W_PER_B
    slot = wid % _W_PER_B
    a0 = slot * _ATOMS_PER_W
    pltpu.sync_copy(d_hbm.at[b, pl.ds(a0, _ATOMS_PER_W), :], d_v)
    pltpu.sync_copy(pt_hbm.at[b, pl.ds(a0, _ATOMS_PER_W), :], pt_v)
    pltpu.sync_copy(ty_hbm.at[b, pl.ds(a0, _ATOMS_PER_W)], ty_v)
    pltpu.sync_copy(par_hbm, par_v)

    pv = par_v[...]

    def bcast(i):
        return jnp.full((_L,), pv[i], jnp.float32)

    lnA = [bcast(0), bcast(1), bcast(2)]
    pp = [bcast(3), bcast(4), bcast(5)]
    lnX = [bcast(6), bcast(7), bcast(8)]
    qq = [bcast(9), bcast(10), bcast(11)]
    d0v, d1v = bcast(12), bcast(13)

    iota = lax.iota(jnp.int32, _L)
    zero = jnp.zeros((_L,), jnp.float32)
    one = jnp.full((_L,), 1, jnp.int32)
    two = jnp.full((_L,), 2, jnp.int32)

    def chunk(a, k, pacc, racc):
        d = d_v[a, pl.ds(k * _L, _L)]
        ptv = pt_v[a, pl.ds(k * _L, _L)]
        m1 = ptv == one
        m2 = ptv == two
        la = jnp.where(m1, lnA[1], jnp.where(m2, lnA[2], lnA[0]))
        p = jnp.where(m1, pp[1], jnp.where(m2, pp[2], pp[0]))
        lx = jnp.where(m1, lnX[1], jnp.where(m2, lnX[2], lnX[0]))
        q = jnp.where(m1, qq[1], jnp.where(m2, qq[2], qq[0]))
        pacc = pacc + jnp.exp(la - p * d)
        racc = racc + jnp.exp(lx - q * d)
        return pacc, racc

    def group_body(g, carry):
        acc_phi, acc_emb = carry

        def atom_body(a, carry2):
            acc2, m = carry2
            pacc, racc = zero, zero
            for k in range(_CHUNKS):
                pacc, racc = chunk(g * _L + a, k, pacc, racc)
            tot = jnp.sum(racc)
            m = jnp.where(iota == a, tot, m)
            return acc2 + pacc, m

        acc_phi, m = lax.fori_loop(0, _L, atom_body, (acc_phi, zero))
        sq = _vsqrt(m)
        tyv = ty_v[pl.ds(pl.multiple_of(g * _L, _L), _L)]
        dsel = jnp.where(tyv == one, d1v, d0v)
        acc_emb = acc_emb - dsel * sq
        return acc_phi, acc_emb

    acc_phi, acc_emb = lax.fori_loop(0, _GROUPS, group_body, (zero, zero))
    res_v[...] = (acc_phi + acc_emb) * jnp.float32(1.0 / _N_ATOMS)
    pltpu.sync_copy(res_v, out_hbm.at[b, pl.ds(slot * _L, _L)])


def _sc_call(d3, pt3, ty2, par):
    mesh = plsc.VectorSubcoreMesh(core_axis_name="c", subcore_axis_name="s")
    run = functools.partial(
        pl.kernel,
        mesh=mesh,
        compiler_params=pltpu.CompilerParams(needs_layout_passes=False),
        out_type=jax.ShapeDtypeStruct((_K_SC, _W_PER_B * _L), jnp.float32),
        scratch_types=[
            pltpu.VMEM((_ATOMS_PER_W, _N_NEIGH), jnp.float32),
            pltpu.VMEM((_ATOMS_PER_W, _N_NEIGH), jnp.int32),
            pltpu.VMEM((_ATOMS_PER_W,), jnp.int32),
            pltpu.VMEM((_L,), jnp.float32),
            pltpu.VMEM((_L,), jnp.float32),
        ],
    )(_sc_body)
    return run(d3, pt3, ty2, par)


def _tc_body(par_ref, d_ref, pt_ref, ty_ref, out_ref):
    d = d_ref[0]
    pt = pt_ref[0]
    m1 = pt == 1
    m2 = pt == 2
    la = jnp.where(m1, par_ref[1], jnp.where(m2, par_ref[2], par_ref[0]))
    p = jnp.where(m1, par_ref[4], jnp.where(m2, par_ref[5], par_ref[3]))
    lx = jnp.where(m1, par_ref[7], jnp.where(m2, par_ref[8], par_ref[6]))
    q = jnp.where(m1, par_ref[10], jnp.where(m2, par_ref[11], par_ref[9]))
    phi = jnp.exp(la - p * d)
    rho = jnp.exp(lx - q * d)
    srho = jnp.sum(rho, axis=-1, keepdims=True)
    ty = ty_ref[0]
    dsel = jnp.where(ty == 1, par_ref[13], par_ref[12])
    emb_tot = jnp.dot(dsel, jnp.sqrt(srho),
                      preferred_element_type=jnp.float32,
                      precision=jax.lax.Precision.HIGHEST)
    tot = (jnp.sum(phi) - emb_tot[0, 0]) * (1.0 / _N_ATOMS)
    out_ref[...] = jnp.full((1, 1, 1), tot, jnp.float32)


def _tc_call(par, d3, pt3, ty3):
    return pl.pallas_call(
        _tc_body,
        grid=(_B_TC,),
        in_specs=[
            pl.BlockSpec(memory_space=pltpu.SMEM),
            pl.BlockSpec((1, _N_ATOMS, _N_NEIGH), lambda b: (b, 0, 0)),
            pl.BlockSpec((1, _N_ATOMS, _N_NEIGH), lambda b: (b, 0, 0)),
            pl.BlockSpec((1, 1, _N_ATOMS), lambda b: (b, 0, 0)),
        ],
        out_specs=pl.BlockSpec((1, 1, 1), lambda b: (b, 0, 0)),
        out_shape=jax.ShapeDtypeStruct((_B_TC, 1, 1), jnp.float32),
    )(par, d3, pt3, ty3)


@jax.jit
def _eam_hybrid(distances, pair_types, types, par):
    tc_e = _tc_call(par, distances, pair_types,
                    types.reshape(_BATCH, 1, _N_ATOMS))
    sc_partials = _sc_call(distances[_B_TC:], pair_types[_B_TC:],
                           types[_B_TC:], par)
    sc_e = sc_partials.sum(axis=1, keepdims=True)
    return jnp.concatenate([tc_e.reshape(_B_TC, 1), sc_e], axis=0)


def kernel(types, distances, pair_types, phi_params, rho_params, emb_params):
    par = jnp.concatenate([
        jnp.log(phi_params[:, 0]), phi_params[:, 1],
        jnp.log(rho_params[:, 0]), rho_params[:, 1],
        emb_params.astype(jnp.float32), jnp.zeros((2,), jnp.float32),
    ]).astype(jnp.float32)
    return _eam_hybrid(distances, pair_types.astype(jnp.int32),
                       types.astype(jnp.int32), par)
